# baseline (speedup 1.0000x reference)
import jax
import jax.numpy as jnp
from jax.experimental import pallas as pl


def _body(hm_ref, h0_ref, out_ref):
    h = h0_ref[...]
    d = h.shape[1]
    out_ref[:, 0:d] = hm_ref[:, 0:d] + h
    out_ref[:, d : 2 * d] = hm_ref[:, d : 2 * d] + h


def kernel(hm, h0):
    n, t, d = hm.shape
    hm2 = hm.reshape(n, t * d)
    R = 2000
    out = pl.pallas_call(
        _body,
        grid=(n // R,),
        in_specs=[
            pl.BlockSpec((R, t * d), lambda i: (i, 0)),
            pl.BlockSpec((R, d), lambda i: (i, 0)),
        ],
        out_specs=pl.BlockSpec((R, t * d), lambda i: (i, 0)),
        out_shape=jax.ShapeDtypeStruct((n, t * d), jnp.float32),
    )(hm2, h0)
    return out.reshape(n, t, d)


# speedup vs baseline: 3.0900x; 3.0900x over previous
import jax
import jax.numpy as jnp
from jax.experimental import pallas as pl


def _body(hm_ref, h0_ref, out_ref):
    h = h0_ref[...]
    out_ref[...] = hm_ref[...] + h[:, None, :]


def kernel(hm, h0):
    n, t, d = hm.shape
    R = 2000
    out = pl.pallas_call(
        _body,
        grid=(n // R,),
        in_specs=[
            pl.BlockSpec((R, t, d), lambda i: (i, 0, 0)),
            pl.BlockSpec((R, d), lambda i: (i, 0)),
        ],
        out_specs=pl.BlockSpec((R, t, d), lambda i: (i, 0, 0)),
        out_shape=jax.ShapeDtypeStruct((n, t, d), jnp.float32),
    )(hm, h0)
    return out


# speedup vs baseline: 3.4654x; 1.1215x over previous
import jax
import jax.numpy as jnp
from jax.experimental import pallas as pl


def _body(hm_ref, h0_ref, out_ref):
    h = h0_ref[...]
    out_ref[...] = hm_ref[...] + h[:, None, :]


def kernel(hm, h0):
    n, t, d = hm.shape
    R = 4000
    out = pl.pallas_call(
        _body,
        grid=(n // R,),
        in_specs=[
            pl.BlockSpec((R, t, d), lambda i: (i, 0, 0)),
            pl.BlockSpec((R, d), lambda i: (i, 0)),
        ],
        out_specs=pl.BlockSpec((R, t, d), lambda i: (i, 0, 0)),
        out_shape=jax.ShapeDtypeStruct((n, t, d), jnp.float32),
    )(hm, h0)
    return out


# speedup vs baseline: 3.5801x; 1.0331x over previous
import jax
import jax.numpy as jnp
from jax.experimental import pallas as pl


def _body(hm_ref, h0_ref, out_ref):
    h = h0_ref[...]
    out_ref[...] = hm_ref[...] + h[:, None, :]


def kernel(hm, h0):
    n, t, d = hm.shape
    R = 10000
    out = pl.pallas_call(
        _body,
        grid=(n // R,),
        in_specs=[
            pl.BlockSpec((R, t, d), lambda i: (i, 0, 0)),
            pl.BlockSpec((R, d), lambda i: (i, 0)),
        ],
        out_specs=pl.BlockSpec((R, t, d), lambda i: (i, 0, 0)),
        out_shape=jax.ShapeDtypeStruct((n, t, d), jnp.float32),
    )(hm, h0)
    return out
